# baseline (speedup 1.0000x reference)
import functools

import jax
import jax.numpy as jnp
from jax import lax
from jax.experimental import pallas as pl
from jax.experimental.pallas import tpu as pltpu
from jax.experimental.pallas import tpu_sc as plsc

_EPS = 1e-12
_LANES = 16
_GSPLIT = 100


def _position_table(seq_len, hidden_size):
    pos = jnp.arange(seq_len, dtype=jnp.float32)[:, None]
    kk = jnp.arange(hidden_size, dtype=jnp.float32)[None, :]
    vals = pos / jnp.power(10000.0, 2.0 * kk / float(hidden_size))
    pe = vals.astype(jnp.int32)
    pe = pe.at[0::2].set(jnp.sin(pe[0::2].astype(jnp.float32)).astype(jnp.int32))
    pe = pe.at[1::2].set(jnp.cos(pe[1::2].astype(jnp.float32)).astype(jnp.int32))
    return pe


def _rsqrt16(v):
    i = plsc.bitcast(v, jnp.int32)
    i = jnp.int32(0x5F3759DF) - lax.shift_right_logical(i, 1)
    y = plsc.bitcast(i, jnp.float32)
    for _ in range(3):
        y = y * (1.5 - 0.5 * v * y * y)
    return y


def _make_sc_kernel(NW, NC, NCH, C, H, V2):
    NG = (C + _LANES - 1) // _LANES
    NSUB = C // _GSPLIT
    inv_h = 1.0 / H

    mesh = plsc.VectorSubcoreMesh(core_axis_name="c", subcore_axis_name="s")

    @functools.partial(
        pl.kernel,
        mesh=mesh,
        compiler_params=pltpu.CompilerParams(
            needs_layout_passes=False, use_tc_tiling_on_sc=True),
        out_type=jax.ShapeDtypeStruct((NW, NCH, C, H), jnp.float32),
        scratch_types=(
            [pltpu.VMEM((NCH, NSUB, _GSPLIT), jnp.int32)]
            + [pltpu.VMEM((NCH, C), jnp.int32)]
            + [pltpu.VMEM((C,), jnp.int32)]
            + [pltpu.VMEM((2, H), jnp.float32)]
            + [pltpu.VMEM((C, 2 * H), jnp.float32) for _ in range(2)]
            + [pltpu.VMEM((C, H), jnp.float32) for _ in range(2)]
            + [pltpu.VMEM((H, _LANES), jnp.float32)]
            + [pltpu.SemaphoreType.DMA for _ in range(4)]
        ),
    )
    def sc_kernel(idx_hbm, table_hbm, par_hbm, thr_hbm, wb_hbm, out_hbm,
                  idx_v, par_v, thr_v, wb_v, i0, i1, o0, o1, tb,
                  g0, g1, s0, s1):
        ibufs = (i0, i1)
        obufs = (o0, o1)
        gsems = (g0, g1)
        osems = (s0, s1)

        wid = lax.axis_index("s") * NC + lax.axis_index("c")
        pltpu.sync_copy(idx_hbm.at[wid], idx_v)
        pltpu.sync_copy(par_hbm.at[wid], par_v)
        pltpu.sync_copy(thr_hbm, thr_v)
        pltpu.sync_copy(wb_hbm, wb_v)

        iota16 = lax.iota(jnp.int32, _LANES)

        def gstart(jn, b):
            for k in range(NSUB):
                pltpu.async_copy(
                    table_hbm.at[idx_v.at[jn, k]],
                    ibufs[b].at[pl.ds(k * _GSPLIT, _GSPLIT)],
                    gsems[b])

        def gwait(jn, b):
            for k in range(NSUB):
                pltpu.make_async_copy(
                    table_hbm.at[idx_v.at[jn, k]],
                    ibufs[b].at[pl.ds(k * _GSPLIT, _GSPLIT)],
                    gsems[b]).wait()

        def ostart(j, b):
            pltpu.async_copy(obufs[b], out_hbm.at[wid, j], osems[b])

        def owait(j, b):
            pltpu.make_async_copy(
                obufs[b], out_hbm.at[wid, j], osems[b]).wait()

        def compute(j, b):
            ibuf, obuf = ibufs[b], obufs[b]

            def group(g, carry):
                r0 = jnp.minimum(g * _LANES, C - _LANES)
                rows = r0 + iota16
                thr = thr_v[pl.ds(r0, _LANES)]
                par = par_v[j, pl.ds(r0, _LANES)]
                cols = [jnp.bitwise_and(iota16 + c, H - 1) for c in range(H)]
                icols = [cols[c] + par for c in range(H)]
                acc1 = [jnp.zeros((_LANES,), jnp.float32) for _ in range(4)]
                acc2 = [jnp.zeros((_LANES,), jnp.float32) for _ in range(4)]
                for c0 in range(0, H, 8):
                    xs = [
                        plsc.load_gather(ibuf, [rows, icols[c0 + i]])
                        for i in range(8)
                    ]
                    for i in range(8):
                        c = c0 + i
                        x = xs[i] + jnp.where(thr <= cols[c], 1.0, 0.0)
                        acc1[i % 4] = acc1[i % 4] + x
                        acc2[i % 4] = acc2[i % 4] + x * x
                        tb[c, :] = x
                s1v = (acc1[0] + acc1[1]) + (acc1[2] + acc1[3])
                s2v = (acc2[0] + acc2[1]) + (acc2[2] + acc2[3])
                u = s1v * inv_h
                var = s2v * inv_h - u * u + _EPS
                y = _rsqrt16(var)
                for c0 in range(0, H, 8):
                    ts = [tb[c0 + i, :] for i in range(8)]
                    ws = [plsc.load_gather(wb_v.at[0], [cols[c0 + i]])
                          for i in range(8)]
                    bs = [plsc.load_gather(wb_v.at[1], [cols[c0 + i]])
                          for i in range(8)]
                    for i in range(8):
                        o = ((ts[i] - u) * y) * ws[i] + bs[i]
                        plsc.store_scatter(obuf, [rows, cols[c0 + i]], o)
                return carry

            lax.fori_loop(0, NG, group, 0)

        gstart(0, 0)
        gstart(1, 1)

        def chunk_pair(i, carry):
            for b in range(2):
                j = 2 * i + b
                gwait(j, b)

                @pl.when(j >= 2)
                def _():
                    owait(j - 2, b)

                compute(j, b)
                ostart(j, b)

                @pl.when(j + 2 < NCH)
                def _():
                    gstart(j + 2, b)

            return carry

        lax.fori_loop(0, NCH // 2, chunk_pair, 0)
        owait(NCH - 2, 0)
        owait(NCH - 1, 1)

    return sc_kernel


def kernel(inputs, table, ln_weight, ln_bias):
    B, T = inputs.shape
    V, H = table.shape
    info = plsc.get_sparse_core_info()
    NC, NS = info.num_cores, info.num_subcores
    NW = NC * NS

    N = B * T
    C = T
    assert C % _GSPLIT == 0 and N % (NW * C) == 0 and V % 2 == 0
    NCH = N // (NW * C)
    assert NCH % 2 == 0

    pe = _position_table(T, H)
    thr = (H - jnp.sum(pe, axis=1)).astype(jnp.int32)

    table2 = jnp.concatenate([table[0::2], table[1::2]], axis=1)
    idxp = (inputs >> 1).reshape(NW, NCH, C // _GSPLIT, _GSPLIT)
    par = ((inputs & 1) * H).reshape(NW, NCH, C)
    wb = jnp.stack([ln_weight, ln_bias])

    f = _make_sc_kernel(NW, NC, NCH, C, H, V // 2)
    out = f(idxp, table2, par, thr, wb)
    return out.reshape(B, T, H)


# speedup vs baseline: 11.0189x; 11.0189x over previous
import functools

import jax
import jax.numpy as jnp
from jax import lax
from jax.experimental import pallas as pl
from jax.experimental.pallas import tpu as pltpu
from jax.experimental.pallas import tpu_sc as plsc

_EPS = 1e-12
_LANES = 16
_GSPLIT = 100


def _position_table(seq_len, hidden_size):
    pos = jnp.arange(seq_len, dtype=jnp.float32)[:, None]
    kk = jnp.arange(hidden_size, dtype=jnp.float32)[None, :]
    vals = pos / jnp.power(10000.0, 2.0 * kk / float(hidden_size))
    pe = vals.astype(jnp.int32)
    pe = pe.at[0::2].set(jnp.sin(pe[0::2].astype(jnp.float32)).astype(jnp.int32))
    pe = pe.at[1::2].set(jnp.cos(pe[1::2].astype(jnp.float32)).astype(jnp.int32))
    return pe


def _rsqrt16(v):
    i = plsc.bitcast(v, jnp.int32)
    i = jnp.int32(0x5F3759DF) - lax.shift_right_logical(i, 1)
    y = plsc.bitcast(i, jnp.float32)
    for _ in range(3):
        y = y * (1.5 - 0.5 * v * y * y)
    return y


def _make_sc_kernel(NW, NC, NCH, C, H, V2):
    NG = (C + _LANES - 1) // _LANES
    NSUB = C // _GSPLIT
    inv_h = 1.0 / H

    mesh = plsc.VectorSubcoreMesh(core_axis_name="c", subcore_axis_name="s")

    @functools.partial(
        pl.kernel,
        mesh=mesh,
        compiler_params=pltpu.CompilerParams(
            needs_layout_passes=False, use_tc_tiling_on_sc=True),
        out_type=jax.ShapeDtypeStruct((NW, NCH, C, H), jnp.float32),
        scratch_types=(
            [pltpu.VMEM((NCH, NSUB, _GSPLIT), jnp.int32)]
            + [pltpu.VMEM((NCH, C), jnp.int32)]
            + [pltpu.VMEM((C,), jnp.int32)]
            + [pltpu.VMEM((2, H), jnp.float32)]
            + [pltpu.VMEM((C, 2 * H), jnp.float32) for _ in range(2)]
            + [pltpu.VMEM((C, H), jnp.float32) for _ in range(2)]
            + [pltpu.VMEM((H, _LANES), jnp.float32)]
            + [pltpu.SemaphoreType.DMA for _ in range(4)]
        ),
    )
    def sc_kernel(idx_hbm, table_hbm, par_hbm, thr_hbm, wb_hbm, out_hbm,
                  idx_v, par_v, thr_v, wb_v, i0, i1, o0, o1, tb,
                  g0, g1, s0, s1):
        ibufs = (i0, i1)
        obufs = (o0, o1)
        gsems = (g0, g1)
        osems = (s0, s1)

        wid = lax.axis_index("s") * NC + lax.axis_index("c")
        pltpu.sync_copy(idx_hbm.at[wid], idx_v)
        pltpu.sync_copy(par_hbm.at[wid], par_v)
        pltpu.sync_copy(thr_hbm, thr_v)
        pltpu.sync_copy(wb_hbm, wb_v)

        iota16 = lax.iota(jnp.int32, _LANES)

        def gstart(jn, b):
            for k in range(NSUB):
                pltpu.async_copy(
                    table_hbm.at[idx_v.at[jn, k]],
                    ibufs[b].at[pl.ds(k * _GSPLIT, _GSPLIT)],
                    gsems[b])

        def gwait(jn, b):
            for k in range(NSUB):
                pltpu.make_async_copy(
                    table_hbm.at[idx_v.at[jn, k]],
                    ibufs[b].at[pl.ds(k * _GSPLIT, _GSPLIT)],
                    gsems[b]).wait()

        def ostart(j, b):
            pltpu.async_copy(obufs[b], out_hbm.at[wid, j], osems[b])

        def owait(j, b):
            pltpu.make_async_copy(
                obufs[b], out_hbm.at[wid, j], osems[b]).wait()

        cols = [jnp.bitwise_and(iota16 + c, H - 1) for c in range(H)]

        def compute(j, b):
            ibuf, obuf = ibufs[b], obufs[b]

            def group(g, carry):
                r0 = jnp.minimum(g * _LANES, C - _LANES)
                rows = r0 + iota16
                thr = thr_v[pl.ds(r0, _LANES)]
                par = par_v[j, pl.ds(r0, _LANES)]
                icols = [cols[c] + par for c in range(H)]
                acc1 = [jnp.zeros((_LANES,), jnp.float32) for _ in range(4)]
                acc2 = [jnp.zeros((_LANES,), jnp.float32) for _ in range(4)]
                for c0 in range(0, H, 8):
                    xs = [
                        plsc.load_gather(ibuf, [rows, icols[c0 + i]])
                        for i in range(8)
                    ]
                    for i in range(8):
                        c = c0 + i
                        x = xs[i] + jnp.where(thr <= cols[c], 1.0, 0.0)
                        acc1[i % 4] = acc1[i % 4] + x
                        acc2[i % 4] = acc2[i % 4] + x * x
                        tb[c, :] = x
                s1v = (acc1[0] + acc1[1]) + (acc1[2] + acc1[3])
                s2v = (acc2[0] + acc2[1]) + (acc2[2] + acc2[3])
                u = s1v * inv_h
                var = s2v * inv_h - u * u + _EPS
                y = _rsqrt16(var)
                for c0 in range(0, H, 8):
                    ts = [tb[c0 + i, :] for i in range(8)]
                    ws = [plsc.load_gather(wb_v.at[0], [cols[c0 + i]])
                          for i in range(8)]
                    bs = [plsc.load_gather(wb_v.at[1], [cols[c0 + i]])
                          for i in range(8)]
                    for i in range(8):
                        o = ((ts[i] - u) * y) * ws[i] + bs[i]
                        plsc.store_scatter(obuf, [rows, cols[c0 + i]], o)
                return carry

            lax.fori_loop(0, NG, group, 0)

        gstart(0, 0)
        gstart(1, 1)

        def chunk_pair(i, carry):
            for b in range(2):
                j = 2 * i + b
                gwait(j, b)

                @pl.when(j >= 2)
                def _():
                    owait(j - 2, b)

                compute(j, b)
                ostart(j, b)

                @pl.when(j + 2 < NCH)
                def _():
                    gstart(j + 2, b)

            return carry

        lax.fori_loop(0, NCH // 2, chunk_pair, 0)
        owait(NCH - 2, 0)
        owait(NCH - 1, 1)

    return sc_kernel


def kernel(inputs, table, ln_weight, ln_bias):
    B, T = inputs.shape
    V, H = table.shape
    info = plsc.get_sparse_core_info()
    NC, NS = info.num_cores, info.num_subcores
    NW = NC * NS

    N = B * T
    C = T
    assert C % _GSPLIT == 0 and N % (NW * C) == 0 and V % 2 == 0
    NCH = N // (NW * C)
    assert NCH % 2 == 0

    pe = _position_table(T, H)
    thr = (H - jnp.sum(pe, axis=1)).astype(jnp.int32)

    table2 = table.reshape(V // 2, 2 * H)
    idxp = (inputs >> 1).reshape(NW, NCH, C // _GSPLIT, _GSPLIT)
    par = ((inputs & 1) * H).reshape(NW, NCH, C)
    wb = jnp.stack([ln_weight, ln_bias])

    f = _make_sc_kernel(NW, NC, NCH, C, H, V // 2)
    out = f(idxp, table2, par, thr, wb)
    return out.reshape(B, T, H)


# speedup vs baseline: 12.3270x; 1.1187x over previous
import functools

import jax
import jax.numpy as jnp
from jax import lax
from jax.experimental import pallas as pl
from jax.experimental.pallas import tpu as pltpu
from jax.experimental.pallas import tpu_sc as plsc

_EPS = 1e-12
_LANES = 16
_GSPLIT = 100


def _position_table(seq_len, hidden_size):
    pos = jnp.arange(seq_len, dtype=jnp.float32)[:, None]
    kk = jnp.arange(hidden_size, dtype=jnp.float32)[None, :]
    vals = pos / jnp.power(10000.0, 2.0 * kk / float(hidden_size))
    pe = vals.astype(jnp.int32)
    pe = pe.at[0::2].set(jnp.sin(pe[0::2].astype(jnp.float32)).astype(jnp.int32))
    pe = pe.at[1::2].set(jnp.cos(pe[1::2].astype(jnp.float32)).astype(jnp.int32))
    return pe


def _rsqrt16(v):
    i = plsc.bitcast(v, jnp.int32)
    i = jnp.int32(0x5F3759DF) - lax.shift_right_logical(i, 1)
    y = plsc.bitcast(i, jnp.float32)
    for _ in range(3):
        y = y * (1.5 - 0.5 * v * y * y)
    return y


def _make_sc_kernel(NW, NC, NCH, C, H, V2):
    NG = (C + _LANES - 1) // _LANES
    NSUB = C // _GSPLIT
    inv_h = 1.0 / H

    mesh = plsc.VectorSubcoreMesh(core_axis_name="c", subcore_axis_name="s")

    @functools.partial(
        pl.kernel,
        mesh=mesh,
        compiler_params=pltpu.CompilerParams(
            needs_layout_passes=False, use_tc_tiling_on_sc=True),
        out_type=jax.ShapeDtypeStruct((NW, NCH, C, H), jnp.float32),
        scratch_types=(
            [pltpu.VMEM((NCH, NSUB, _GSPLIT), jnp.int32)]
            + [pltpu.VMEM((NCH, C), jnp.int32)]
            + [pltpu.VMEM((C,), jnp.int32)]
            + [pltpu.VMEM((2, H), jnp.float32)]
            + [pltpu.VMEM((C, 2 * H), jnp.float32) for _ in range(2)]
            + [pltpu.VMEM((C, H), jnp.float32) for _ in range(2)]
            + [pltpu.VMEM((H, _LANES), jnp.float32)]
            + [pltpu.SemaphoreType.DMA for _ in range(4)]
        ),
    )
    def sc_kernel(idx_hbm, table_hbm, par_hbm, thr_hbm, wb_hbm, out_hbm,
                  idx_v, par_v, thr_v, wb_v, i0, i1, o0, o1, tb,
                  g0, g1, s0, s1):
        ibufs = (i0, i1)
        obufs = (o0, o1)
        gsems = (g0, g1)
        osems = (s0, s1)

        wid = lax.axis_index("s") * NC + lax.axis_index("c")
        pltpu.sync_copy(idx_hbm.at[wid], idx_v)
        pltpu.sync_copy(par_hbm.at[wid], par_v)
        pltpu.sync_copy(thr_hbm, thr_v)
        pltpu.sync_copy(wb_hbm, wb_v)

        iota16 = lax.iota(jnp.int32, _LANES)

        def gstart(jn, b):
            for k in range(NSUB):
                pltpu.async_copy(
                    table_hbm.at[idx_v.at[jn, k]],
                    ibufs[b].at[pl.ds(k * _GSPLIT, _GSPLIT)],
                    gsems[b])

        def gwait(jn, b):
            for k in range(NSUB):
                pltpu.make_async_copy(
                    table_hbm.at[idx_v.at[jn, k]],
                    ibufs[b].at[pl.ds(k * _GSPLIT, _GSPLIT)],
                    gsems[b]).wait()

        def ostart(j, b):
            pltpu.async_copy(obufs[b], out_hbm.at[wid, j], osems[b])

        def owait(j, b):
            pltpu.make_async_copy(
                obufs[b], out_hbm.at[wid, j], osems[b]).wait()

        cols = [jnp.bitwise_and(iota16 + c, H - 1) for c in range(H)]

        def compute(j, b):
            ibuf, obuf = ibufs[b], obufs[b]

            def group(g, carry):
                r0 = jnp.minimum(g * _LANES, C - _LANES)
                rows = r0 + iota16
                thr = thr_v[pl.ds(r0, _LANES)]
                par = par_v[j, pl.ds(r0, _LANES)]
                icols = [cols[c] + par for c in range(H)]
                acc1 = [jnp.zeros((_LANES,), jnp.float32) for _ in range(4)]
                acc2 = [jnp.zeros((_LANES,), jnp.float32) for _ in range(4)]
                for c0 in range(0, H, 8):
                    xs = [
                        plsc.load_gather(ibuf, [rows, icols[c0 + i]])
                        for i in range(8)
                    ]
                    for i in range(8):
                        c = c0 + i
                        x = xs[i] + jnp.where(thr <= cols[c], 1.0, 0.0)
                        acc1[i % 4] = acc1[i % 4] + x
                        acc2[i % 4] = acc2[i % 4] + x * x
                        tb[c, :] = x
                s1v = (acc1[0] + acc1[1]) + (acc1[2] + acc1[3])
                s2v = (acc2[0] + acc2[1]) + (acc2[2] + acc2[3])
                u = s1v * inv_h
                var = s2v * inv_h - u * u + _EPS
                y = _rsqrt16(var)
                for c0 in range(0, H, 8):
                    ts = [tb[c0 + i, :] for i in range(8)]
                    ws = [plsc.load_gather(wb_v.at[0], [cols[c0 + i]])
                          for i in range(8)]
                    bs = [plsc.load_gather(wb_v.at[1], [cols[c0 + i]])
                          for i in range(8)]
                    for i in range(8):
                        o = ((ts[i] - u) * y) * ws[i] + bs[i]
                        plsc.store_scatter(obuf, [rows, cols[c0 + i]], o)
                return carry

            lax.fori_loop(0, NG, group, 0)

        gstart(0, 0)
        gstart(1, 1)

        def chunk_pair(i, carry):
            for b in range(2):
                j = 2 * i + b
                gwait(j, b)

                @pl.when(j >= 2)
                def _():
                    owait(j - 2, b)

                ostart(j, b)

                @pl.when(j + 2 < NCH)
                def _():
                    gstart(j + 2, b)

            return carry

        lax.fori_loop(0, NCH // 2, chunk_pair, 0)
        owait(NCH - 2, 0)
        owait(NCH - 1, 1)

    return sc_kernel


def kernel(inputs, table, ln_weight, ln_bias):
    B, T = inputs.shape
    V, H = table.shape
    info = plsc.get_sparse_core_info()
    NC, NS = info.num_cores, info.num_subcores
    NW = NC * NS

    N = B * T
    C = T
    assert C % _GSPLIT == 0 and N % (NW * C) == 0 and V % 2 == 0
    NCH = N // (NW * C)
    assert NCH % 2 == 0

    pe = _position_table(T, H)
    thr = (H - jnp.sum(pe, axis=1)).astype(jnp.int32)

    table2 = table.reshape(V // 2, 2 * H)
    idxp = (inputs >> 1).reshape(NW, NCH, C // _GSPLIT, _GSPLIT)
    par = ((inputs & 1) * H).reshape(NW, NCH, C)
    wb = jnp.stack([ln_weight, ln_bias])

    f = _make_sc_kernel(NW, NC, NCH, C, H, V // 2)
    out = f(idxp, table2, par, thr, wb)
    return out.reshape(B, T, H)
